# baseline (speedup 1.0000x reference)
import functools

import jax
import jax.numpy as jnp
from jax import lax
from jax.experimental import pallas as pl
from jax.experimental.pallas import tpu as pltpu
from jax.experimental.pallas import tpu_sc as plsc

N = 10000
E = 320000
D = 128
NC = 2
NS = 16
NW = NC * NS
CHUNK = 128
NB = 2
CH = 80
EPAD = NW * CH * CHUNK
NPAD = 10240
RPT = NPAD // NS
BLK = 1024
GRID = NPAD // BLK

_mesh = plsc.VectorSubcoreMesh(core_axis_name="c", subcore_axis_name="s")


@functools.partial(
    pl.kernel,
    out_type=jax.ShapeDtypeStruct((NC, NPAD, D), jnp.float32),
    mesh=_mesh,
    scratch_types=(
        [pltpu.VMEM((CHUNK,), jnp.int32) for _ in range(NB)]
        + [pltpu.VMEM((CHUNK,), jnp.int32) for _ in range(NB)]
        + [pltpu.VMEM((CHUNK, D), jnp.float32) for _ in range(NB)]
        + [pltpu.SemaphoreType.DMA for _ in range(2 * NB)]
        + [pltpu.VMEM_SHARED((NPAD, D), jnp.float32)]
    ),
)
def _sc_spmm(g_hbm, src_hbm, dst_hbm, zeros_hbm, out, *scr):
    srcb = scr[0:NB]
    dstb = scr[NB:2 * NB]
    rows = scr[2 * NB:3 * NB]
    gsem = scr[3 * NB:4 * NB]
    ssem = scr[4 * NB:5 * NB]
    acc_sh = scr[5 * NB]
    c = lax.axis_index("c")
    s = lax.axis_index("s")
    wid = c * NS + s
    pltpu.sync_copy(zeros_hbm.at[pl.ds(s * RPT, RPT)],
                    acc_sh.at[pl.ds(s * RPT, RPT)])
    plsc.subcore_barrier()

    @pl.loop(0, CH, step=NB)
    def _outer(j):
        handles = []
        for b in range(NB):
            @pl.when(j > 0)
            def _():
                pltpu.make_async_copy(rows[b], acc_sh.at[dstb[b]],
                                      ssem[b]).wait()
            pltpu.sync_copy(src_hbm.at[wid, j + b], srcb[b])
            pltpu.sync_copy(dst_hbm.at[wid, j + b], dstb[b])
            handles.append(pltpu.async_copy(g_hbm.at[srcb[b]], rows[b],
                                            gsem[b]))
        for b in range(NB):
            handles[b].wait()
            pltpu.async_copy(rows[b], acc_sh.at[dstb[b]], ssem[b], add=True)

    for b in range(NB):
        pltpu.make_async_copy(rows[b], acc_sh.at[dstb[b]], ssem[b]).wait()
    plsc.subcore_barrier()
    pltpu.sync_copy(acc_sh.at[pl.ds(s * RPT, RPT)],
                    out.at[c, pl.ds(s * RPT, RPT)])


@functools.partial(
    pl.kernel,
    out_type=jax.ShapeDtypeStruct((NC, NPAD, D), jnp.float32),
    mesh=_mesh,
    scratch_types=(
        [pltpu.VMEM((CHUNK,), jnp.int32) for _ in range(NB)]
        + [pltpu.VMEM((CHUNK, D), jnp.float32)]
        + [pltpu.SemaphoreType.DMA for _ in range(NB)]
        + [pltpu.VMEM_SHARED((NPAD, D), jnp.float32)]
    ),
)
def _sc_degree(dst_hbm, ones_hbm, zeros_hbm, out, *scr):
    dstb = scr[0:NB]
    ones_v = scr[NB]
    ssem = scr[NB + 1:2 * NB + 1]
    acc_sh = scr[2 * NB + 1]
    c = lax.axis_index("c")
    s = lax.axis_index("s")
    wid = c * NS + s
    pltpu.sync_copy(zeros_hbm.at[pl.ds(s * RPT, RPT)],
                    acc_sh.at[pl.ds(s * RPT, RPT)])
    pltpu.sync_copy(ones_hbm, ones_v)
    plsc.subcore_barrier()

    @pl.loop(0, CH, step=NB)
    def _outer(j):
        for b in range(NB):
            @pl.when(j > 0)
            def _():
                pltpu.make_async_copy(ones_v, acc_sh.at[dstb[b]],
                                      ssem[b]).wait()
            pltpu.sync_copy(dst_hbm.at[wid, j + b], dstb[b])
            pltpu.async_copy(ones_v, acc_sh.at[dstb[b]], ssem[b], add=True)

    for b in range(NB):
        pltpu.make_async_copy(ones_v, acc_sh.at[dstb[b]], ssem[b]).wait()
    plsc.subcore_barrier()
    pltpu.sync_copy(acc_sh.at[pl.ds(s * RPT, RPT)],
                    out.at[c, pl.ds(s * RPT, RPT)])


def _prep_body(dp0, dp1, x, normb_o, g0_o):
    deg = dp0[0][:, :1] + dp1[0][:, :1]
    norm = lax.rsqrt(jnp.maximum(deg, 1.0))
    normb = jnp.broadcast_to(norm, (BLK, D))
    normb_o[...] = normb
    g0_o[...] = x[...] * normb


def _mid_body(normb, p0, p1, x1_o, g1_o):
    nb = normb[...]
    x1 = -(nb * (p0[0] + p1[0]))
    x1_o[...] = x1
    g1_o[...] = nb * x1


def _layer_body(normb, x, x1, q0, q1, W, b, h_o, g0n_o):
    nb = normb[...]
    x2 = -2.0 * nb * (q0[0] + q1[0]) - x[...]
    acc = (jnp.dot(x[...], W[0], preferred_element_type=jnp.float32)
           + jnp.dot(x1[...], W[1], preferred_element_type=jnp.float32)
           + jnp.dot(x2, W[2], preferred_element_type=jnp.float32)
           + b[...])
    h = jnp.maximum(acc, 0.0)
    h_o[...] = h
    g0n_o[...] = nb * h


def _final_body(normb, x, x1, q0, q1, W, b, Wm1, bm1, Wm2, bm2, out_o):
    nb = normb[...]
    x2 = -2.0 * nb * (q0[0] + q1[0]) - x[...]
    acc = (jnp.dot(x[...], W[0], preferred_element_type=jnp.float32)
           + jnp.dot(x1[...], W[1], preferred_element_type=jnp.float32)
           + jnp.dot(x2, W[2], preferred_element_type=jnp.float32)
           + b[...])
    h2 = jnp.maximum(acc, 0.0)
    h3 = jnp.maximum(jnp.dot(h2, Wm1[...], preferred_element_type=jnp.float32)
                     + bm1[...], 0.0)
    out_o[...] = (jnp.dot(h3, Wm2[...], preferred_element_type=jnp.float32)
                  + bm2[...])


def _row_spec():
    return pl.BlockSpec((BLK, D), lambda i: (i, 0))


def _plane_spec(plane):
    return pl.BlockSpec((1, BLK, D), lambda i, p=plane: (p, i, 0))


def _full_spec(shape):
    nd = len(shape)
    return pl.BlockSpec(shape, lambda i: (0,) * nd)


def _tc_call(body, in_specs, out_specs, out_shape):
    return pl.pallas_call(
        body,
        grid=(GRID,),
        in_specs=in_specs,
        out_specs=out_specs,
        out_shape=out_shape,
    )


def kernel(edge_index, features, W1, b1, W2, b2, Wm1, bm1, Wm2, bm2):
    src = edge_index[0]
    dst = edge_index[1]
    pad = EPAD - E
    src3 = jnp.concatenate(
        [src, jnp.zeros((pad,), jnp.int32)]).reshape(NW, CH, CHUNK)
    dst3 = jnp.concatenate(
        [dst, jnp.full((pad,), N, jnp.int32)]).reshape(NW, CH, CHUNK)
    zeros_big = jnp.zeros((NPAD, D), jnp.float32)
    ones_big = jnp.ones((CHUNK, D), jnp.float32)
    b1r = b1.reshape(1, D)
    b2r = b2.reshape(1, D)
    bm1r = bm1.reshape(1, D)
    bm2r = bm2.reshape(1, D)

    dp = _sc_degree(dst3, ones_big, zeros_big)

    rowD = jax.ShapeDtypeStruct((NPAD, D), jnp.float32)

    normb, g0 = _tc_call(
        _prep_body,
        [_plane_spec(0), _plane_spec(1), _row_spec()],
        (_row_spec(), _row_spec()), (rowD, rowD))(dp, dp, features)

    def cheb_mid(g):
        p = _sc_spmm(g, src3, dst3, zeros_big)
        x1, g1 = _tc_call(
            _mid_body,
            [_row_spec(), _plane_spec(0), _plane_spec(1)],
            (_row_spec(), _row_spec()), (rowD, rowD))(normb, p, p)
        q = _sc_spmm(g1, src3, dst3, zeros_big)
        return x1, q

    x1, q = cheb_mid(g0)
    h1, g0b = _tc_call(
        _layer_body,
        [_row_spec(), _row_spec(), _row_spec(), _plane_spec(0),
         _plane_spec(1), _full_spec((3, D, D)), _full_spec((1, D))],
        (_row_spec(), _row_spec()), (rowD, rowD))(
            normb, features, x1, q, q, W1, b1r)

    x1b, qb = cheb_mid(g0b)
    out = _tc_call(
        _final_body,
        [_row_spec(), _row_spec(), _row_spec(), _plane_spec(0),
         _plane_spec(1), _full_spec((3, D, D)), _full_spec((1, D)),
         _full_spec((D, D)), _full_spec((1, D)), _full_spec((D, D)),
         _full_spec((1, D))],
        _row_spec(), jax.ShapeDtypeStruct((N, D), jnp.float32))(
            normb, h1, x1b, qb, qb, W2, b2r, Wm1, bm1r, Wm2, bm2r)
    return out


# speedup vs baseline: 1.2765x; 1.2765x over previous
import functools

import jax
import jax.numpy as jnp
from jax import lax
from jax.experimental import pallas as pl
from jax.experimental.pallas import tpu as pltpu
from jax.experimental.pallas import tpu_sc as plsc

N = 10000
E = 320000
D = 128
NC = 2
NS = 16
NW = NC * NS
CHUNK = 128
NB = 2
CH = 80
CH0 = 124
CH1 = 36
EPAD = NW * CH * CHUNK
NPAD = 10240
RPT = NPAD // NS
BLK = 1024
GRID = NPAD // BLK

_mesh = plsc.VectorSubcoreMesh(core_axis_name="c", subcore_axis_name="s")


@functools.partial(
    pl.kernel,
    out_type=jax.ShapeDtypeStruct((NC, NPAD, D), jnp.float32),
    mesh=_mesh,
    scratch_types=(
        [pltpu.VMEM((CHUNK,), jnp.int32) for _ in range(NB)]
        + [pltpu.VMEM((CHUNK,), jnp.int32) for _ in range(NB)]
        + [pltpu.VMEM((CHUNK, D), jnp.float32) for _ in range(NB)]
        + [pltpu.SemaphoreType.DMA for _ in range(2 * NB)]
        + [pltpu.VMEM_SHARED((NPAD, D), jnp.float32)]
    ),
)
def _sc_spmm(g_hbm, src_hbm, dst_hbm, zeros_hbm, out, *scr):
    srcb = scr[0:NB]
    dstb = scr[NB:2 * NB]
    rows = scr[2 * NB:3 * NB]
    gsem = scr[3 * NB:4 * NB]
    ssem = scr[4 * NB:5 * NB]
    acc_sh = scr[5 * NB]
    c = lax.axis_index("c")
    s = lax.axis_index("s")
    wid = c * NS + s
    nch = jnp.where(c == 0, CH0, CH1)
    pltpu.sync_copy(zeros_hbm.at[pl.ds(s * RPT, RPT)],
                    acc_sh.at[pl.ds(s * RPT, RPT)])
    plsc.subcore_barrier()

    @pl.loop(0, nch, step=NB)
    def _outer(j):
        handles = []
        for b in range(NB):
            @pl.when(j > 0)
            def _():
                pltpu.make_async_copy(rows[b], acc_sh.at[dstb[b]],
                                      ssem[b]).wait()
            pltpu.sync_copy(src_hbm.at[wid, j + b], srcb[b])
            pltpu.sync_copy(dst_hbm.at[wid, j + b], dstb[b])
            handles.append(pltpu.async_copy(g_hbm.at[srcb[b]], rows[b],
                                            gsem[b]))
        for b in range(NB):
            handles[b].wait()
            pltpu.async_copy(rows[b], acc_sh.at[dstb[b]], ssem[b], add=True)

    for b in range(NB):
        pltpu.make_async_copy(rows[b], acc_sh.at[dstb[b]], ssem[b]).wait()
    plsc.subcore_barrier()
    pltpu.sync_copy(acc_sh.at[pl.ds(s * RPT, RPT)],
                    out.at[c, pl.ds(s * RPT, RPT)])


@functools.partial(
    pl.kernel,
    out_type=jax.ShapeDtypeStruct((NC, NPAD, D), jnp.float32),
    mesh=_mesh,
    scratch_types=(
        [pltpu.VMEM((CHUNK,), jnp.int32) for _ in range(NB)]
        + [pltpu.VMEM((CHUNK, D), jnp.float32)]
        + [pltpu.SemaphoreType.DMA for _ in range(NB)]
        + [pltpu.VMEM_SHARED((NPAD, D), jnp.float32)]
    ),
)
def _sc_degree(dst_hbm, ones_hbm, zeros_hbm, out, *scr):
    dstb = scr[0:NB]
    ones_v = scr[NB]
    ssem = scr[NB + 1:2 * NB + 1]
    acc_sh = scr[2 * NB + 1]
    c = lax.axis_index("c")
    s = lax.axis_index("s")
    wid = c * NS + s
    pltpu.sync_copy(zeros_hbm.at[pl.ds(s * RPT, RPT)],
                    acc_sh.at[pl.ds(s * RPT, RPT)])
    pltpu.sync_copy(ones_hbm, ones_v)
    plsc.subcore_barrier()

    @pl.loop(0, CH, step=NB)
    def _outer(j):
        for b in range(NB):
            @pl.when(j > 0)
            def _():
                pltpu.make_async_copy(ones_v, acc_sh.at[dstb[b]],
                                      ssem[b]).wait()
            pltpu.sync_copy(dst_hbm.at[wid, j + b], dstb[b])
            pltpu.async_copy(ones_v, acc_sh.at[dstb[b]], ssem[b], add=True)

    for b in range(NB):
        pltpu.make_async_copy(ones_v, acc_sh.at[dstb[b]], ssem[b]).wait()
    plsc.subcore_barrier()
    pltpu.sync_copy(acc_sh.at[pl.ds(s * RPT, RPT)],
                    out.at[c, pl.ds(s * RPT, RPT)])


def _prep_body(dp0, dp1, x, normb_o, g0_o):
    deg = dp0[0][:, :1] + dp1[0][:, :1]
    norm = lax.rsqrt(jnp.maximum(deg, 1.0))
    normb = jnp.broadcast_to(norm, (BLK, D))
    normb_o[...] = normb
    g0_o[...] = x[...] * normb


def _mid_body(normb, p0, p1, x1_o, g1_o):
    nb = normb[...]
    x1 = -(nb * (p0[0] + p1[0]))
    x1_o[...] = x1
    g1_o[...] = nb * x1


def _layer_body(normb, x, x1, q0, q1, W, b, h_o, g0n_o):
    nb = normb[...]
    x2 = -2.0 * nb * (q0[0] + q1[0]) - x[...]
    acc = (jnp.dot(x[...], W[0], preferred_element_type=jnp.float32)
           + jnp.dot(x1[...], W[1], preferred_element_type=jnp.float32)
           + jnp.dot(x2, W[2], preferred_element_type=jnp.float32)
           + b[...])
    h = jnp.maximum(acc, 0.0)
    h_o[...] = h
    g0n_o[...] = nb * h


def _final_body(normb, x, x1, q0, q1, W, b, Wm1, bm1, Wm2, bm2, out_o):
    nb = normb[...]
    x2 = -2.0 * nb * (q0[0] + q1[0]) - x[...]
    acc = (jnp.dot(x[...], W[0], preferred_element_type=jnp.float32)
           + jnp.dot(x1[...], W[1], preferred_element_type=jnp.float32)
           + jnp.dot(x2, W[2], preferred_element_type=jnp.float32)
           + b[...])
    h2 = jnp.maximum(acc, 0.0)
    h3 = jnp.maximum(jnp.dot(h2, Wm1[...], preferred_element_type=jnp.float32)
                     + bm1[...], 0.0)
    out_o[...] = (jnp.dot(h3, Wm2[...], preferred_element_type=jnp.float32)
                  + bm2[...])


def _row_spec():
    return pl.BlockSpec((BLK, D), lambda i: (i, 0))


def _plane_spec(plane):
    return pl.BlockSpec((1, BLK, D), lambda i, p=plane: (p, i, 0))


def _full_spec(shape):
    nd = len(shape)
    return pl.BlockSpec(shape, lambda i: (0,) * nd)


def _tc_call(body, in_specs, out_specs, out_shape):
    return pl.pallas_call(
        body,
        grid=(GRID,),
        in_specs=in_specs,
        out_specs=out_specs,
        out_shape=out_shape,
    )


def kernel(edge_index, features, W1, b1, W2, b2, Wm1, bm1, Wm2, bm2):
    src = edge_index[0]
    dst = edge_index[1]
    pad = EPAD - E
    srcp = jnp.concatenate([src, jnp.zeros((pad,), jnp.int32)])
    dstp = jnp.concatenate([dst, jnp.full((pad,), N, jnp.int32)])
    n0 = NS * CH0 * CHUNK
    src3 = jnp.concatenate([
        srcp[:n0].reshape(NS, CH0, CHUNK),
        jnp.pad(srcp[n0:].reshape(NS, CH1, CHUNK),
                ((0, 0), (0, CH0 - CH1), (0, 0))),
    ], axis=0)
    dst3 = jnp.concatenate([
        dstp[:n0].reshape(NS, CH0, CHUNK),
        jnp.pad(dstp[n0:].reshape(NS, CH1, CHUNK),
                ((0, 0), (0, CH0 - CH1), (0, 0)), constant_values=N),
    ], axis=0)
    dstd = dstp.reshape(NW, CH, CHUNK)
    zeros_big = jnp.zeros((NPAD, D), jnp.float32)
    ones_big = jnp.ones((CHUNK, D), jnp.float32)
    b1r = b1.reshape(1, D)
    b2r = b2.reshape(1, D)
    bm1r = bm1.reshape(1, D)
    bm2r = bm2.reshape(1, D)

    dp = _sc_degree(dstd, ones_big, zeros_big)

    rowD = jax.ShapeDtypeStruct((NPAD, D), jnp.float32)

    normb, g0 = _tc_call(
        _prep_body,
        [_plane_spec(0), _plane_spec(1), _row_spec()],
        (_row_spec(), _row_spec()), (rowD, rowD))(dp, dp, features)

    def cheb_mid(g):
        p = _sc_spmm(g, src3, dst3, zeros_big)
        x1, g1 = _tc_call(
            _mid_body,
            [_row_spec(), _plane_spec(0), _plane_spec(1)],
            (_row_spec(), _row_spec()), (rowD, rowD))(normb, p, p)
        q = _sc_spmm(g1, src3, dst3, zeros_big)
        return x1, q

    x1, q = cheb_mid(g0)
    h1, g0b = _tc_call(
        _layer_body,
        [_row_spec(), _row_spec(), _row_spec(), _plane_spec(0),
         _plane_spec(1), _full_spec((3, D, D)), _full_spec((1, D))],
        (_row_spec(), _row_spec()), (rowD, rowD))(
            normb, features, x1, q, q, W1, b1r)

    x1b, qb = cheb_mid(g0b)
    out = _tc_call(
        _final_body,
        [_row_spec(), _row_spec(), _row_spec(), _plane_spec(0),
         _plane_spec(1), _full_spec((3, D, D)), _full_spec((1, D)),
         _full_spec((D, D)), _full_spec((1, D)), _full_spec((D, D)),
         _full_spec((1, D))],
        _row_spec(), jax.ShapeDtypeStruct((N, D), jnp.float32))(
            normb, h1, x1b, qb, qb, W2, b2r, Wm1, bm1r, Wm2, bm2r)
    return out
